# baseline (speedup 1.0000x reference)
import jax
import jax.numpy as jnp
from jax import lax
from jax.experimental import pallas as pl
from jax.experimental.pallas import tpu as pltpu
from jax.experimental.pallas import tpu_sc as plsc

_MU = 0.02
_BINS = 10
_LOSS_WEIGHT = 1.0

_L = 16
_NC = 2
_NS = 16
_NW = _NC * _NS
_BINS_PAD = 16
_HREP = 4
_BW = 131072


def _make_elem_body(n_valid, bw, nb, boff=0):
    mu2 = _MU * _MU

    def body(p_ref, t_ref, w_ref, lv_ref):
        b = pl.program_id(0) + boff
        p = p_ref[...]
        t = t_ref[...]
        w = w_ref[...]
        d = p - t
        x = d * d + mu2
        loss = jnp.sqrt(x) - _MU
        g10 = jnp.abs(d) * lax.rsqrt(x) * 10.0
        bini = jnp.minimum(g10.astype(jnp.int32), _BINS - 1)
        ok = w > 0.0
        lv = jnp.where(ok, loss, 0.0)
        bn = jnp.where(ok, bini, _BINS_PAD - 1)
        packed = (lax.bitcast_convert_type(lv, jnp.int32) & ~15) | bn
        rb = p.size // 128

        @pl.when(b < nb - 1)
        def _():
            lv_ref[...] = packed.reshape(rb, 128)

        @pl.when(b == nb - 1)
        def _():
            colg = b * bw + lax.broadcasted_iota(jnp.int32, p.shape, 1)
            pm = jnp.where(colg < n_valid, packed, _BINS_PAD - 1)
            lv_ref[...] = pm.reshape(rb, 128)

    return body


def _make_hist_body(out_r):
    rows_w = out_r // _NW
    nchunks = 16
    cr = rows_w // nchunks
    hwords = _BINS_PAD * _L

    def body(lv_hbm, cnt_out, sum_out, bufs, hcnts, hsums, sems):
        wid = lax.axis_index("s") * _NC + lax.axis_index("c")
        base = wid * rows_w

        lane = lax.iota(jnp.int32, _L)
        ones = jnp.ones((_L,), jnp.float32)
        zeros = jnp.zeros((_L,), jnp.float32)
        for k in range(_HREP):
            for r in range(_BINS_PAD):
                hcnts[k][pl.ds(r * _L, _L)] = zeros
                hsums[k][pl.ds(r * _L, _L)] = zeros

        def copies(j, slot):
            row0 = base + j * cr
            return pltpu.make_async_copy(
                lv_hbm.at[pl.ds(row0, cr), :], bufs[slot], sems[slot])

        copies(0, 0).start()

        for j in range(nchunks):
            slot = j % 2
            if j + 1 < nchunks:
                copies(j + 1, 1 - slot).start()
            copies(j, slot).wait()
            blv = bufs[slot]

            def step(r, carry):
                vals = []
                for c in range(128 // _L):
                    vb = blv[r, pl.ds(c * _L, _L)]
                    slot_i = ((vb & 15) << 4) + lane
                    lossv = lax.bitcast_convert_type(vb & ~15, jnp.float32)
                    vals.append((c % _HREP, slot_i, lossv))
                for k, slot_i, lossv in vals:
                    plsc.addupdate_scatter(hsums[k], [slot_i], lossv)
                for k, slot_i, _ in vals:
                    plsc.addupdate_scatter(hcnts[k], [slot_i], ones)
                return carry

            lax.fori_loop(0, cr, step, 0, unroll=2)

        for k in range(_HREP):
            pltpu.sync_copy(hcnts[k], cnt_out.at[wid, pl.ds(k * hwords, hwords)])
            pltpu.sync_copy(hsums[k], sum_out.at[wid, pl.ds(k * hwords, hwords)])

    return body


def _epilogue_body(cnt_a, sum_a, cnt_b, sum_b, out_ref):
    c = jnp.sum(cnt_a[...], axis=0) + jnp.sum(cnt_b[...], axis=0)
    s = jnp.sum(sum_a[...], axis=0) + jnp.sum(sum_b[...], axis=0)
    cb = jnp.sum(c, axis=1, keepdims=True)
    sb = jnp.sum(s, axis=1, keepdims=True)
    rowid = lax.broadcasted_iota(jnp.int32, cb.shape, 0)
    nz = (cb > 0.0) & (rowid < _BINS)
    n = jnp.sum(nz.astype(jnp.float32))
    denom = jnp.where(nz, cb * n, 1.0)
    contrib = jnp.where(nz, sb / denom, 0.0)
    total = jnp.sum(contrib, keepdims=True) * jnp.float32(_LOSS_WEIGHT)
    out_ref[...] = total.reshape(1, 1)


def kernel(pred, target, label_weight):
    n_rows, ncol = pred.shape
    nb = -(-n_rows // _BW)
    rb = ncol * _BW // 128
    while (nb * rb) % (2 * _NW * 16):
        nb += 1
    nbh = nb // 2
    assert nb == 2 * nbh
    out_rh = nbh * rb

    xt = pred.T
    tt = target.T
    wt = label_weight.T

    def elem_call(boff):
        return pl.pallas_call(
            _make_elem_body(n_rows, _BW, nb, boff),
            grid=(nbh,),
            in_specs=[pl.BlockSpec((ncol, _BW),
                                   lambda b, o=boff: (0, b + o))] * 3,
            out_specs=[pl.BlockSpec((rb, 128), lambda b: (b, 0))],
            out_shape=[jax.ShapeDtypeStruct((out_rh, 128), jnp.int32)],
        )(xt, tt, wt)[0]

    cr = out_rh // _NW // 16
    mesh = plsc.VectorSubcoreMesh(core_axis_name="c", subcore_axis_name="s")

    def hist_call(packed):
        hist = pl.kernel(
            _make_hist_body(out_rh),
            out_type=(
                jax.ShapeDtypeStruct((_NW, _HREP * _BINS_PAD * _L),
                                     jnp.float32),
                jax.ShapeDtypeStruct((_NW, _HREP * _BINS_PAD * _L),
                                     jnp.float32),
            ),
            mesh=mesh,
            scratch_types=(
                tuple(pltpu.VMEM((cr, 128), jnp.int32) for _ in range(2)),
                tuple(pltpu.VMEM((_BINS_PAD * _L,), jnp.float32)
                      for _ in range(_HREP)),
                tuple(pltpu.VMEM((_BINS_PAD * _L,), jnp.float32)
                      for _ in range(_HREP)),
                tuple(pltpu.SemaphoreType.DMA for _ in range(2)),
            ),
            compiler_params=pltpu.CompilerParams(
                needs_layout_passes=False, use_tc_tiling_on_sc=False),
        )
        cnt, sums = hist(packed)
        return (cnt.reshape(_NW * _HREP, _BINS_PAD, _L),
                sums.reshape(_NW * _HREP, _BINS_PAD, _L))

    cnt_a, sums_a = hist_call(elem_call(0))
    cnt_b, sums_b = hist_call(elem_call(nbh))

    out = pl.pallas_call(
        _epilogue_body,
        out_shape=jax.ShapeDtypeStruct((1, 1), jnp.float32),
    )(cnt_a, sums_a, cnt_b, sums_b)
    return out[0, 0]


# speedup vs baseline: 1.0627x; 1.0627x over previous
import jax
import jax.numpy as jnp
from jax import lax
from jax.experimental import pallas as pl
from jax.experimental.pallas import tpu as pltpu
from jax.experimental.pallas import tpu_sc as plsc

_MU = 0.02
_BINS = 10
_LOSS_WEIGHT = 1.0

_L = 16
_NC = 2
_NS = 16
_NW = _NC * _NS
_BINS_PAD = 16
_HREP = 4
_BW = 131072


def _make_elem_body(n_valid, bw, nb, boff=0):
    mu2 = _MU * _MU

    def body(p_ref, t_ref, w_ref, lv_ref):
        b = pl.program_id(0) + boff
        p = p_ref[...]
        t = t_ref[...]
        w = w_ref[...]
        d = p - t
        x = d * d + mu2
        loss = jnp.sqrt(x) - _MU
        g10 = jnp.abs(d) * lax.rsqrt(x) * 10.0
        bini = jnp.minimum(g10.astype(jnp.int32), _BINS - 1)
        pk = (lax.bitcast_convert_type(loss, jnp.int32) & ~15) | bini
        packed = jnp.where(w > 0.0, pk, _BINS_PAD - 1)
        rb = p.size // 128

        @pl.when(b < nb - 1)
        def _():
            lv_ref[...] = packed.reshape(rb, 128)

        @pl.when(b == nb - 1)
        def _():
            colg = b * bw + lax.broadcasted_iota(jnp.int32, p.shape, 1)
            pm = jnp.where(colg < n_valid, packed, _BINS_PAD - 1)
            lv_ref[...] = pm.reshape(rb, 128)

    return body


def _make_hist_body(out_r):
    rows_w = out_r // _NW
    nchunks = 16
    cr = rows_w // nchunks
    hwords = _BINS_PAD * _L

    def body(lv_hbm, cnt_out, sum_out, bufs, hcnts, hsums, sems):
        wid = lax.axis_index("s") * _NC + lax.axis_index("c")
        base = wid * rows_w

        lane = lax.iota(jnp.int32, _L)
        ones = jnp.ones((_L,), jnp.float32)
        zeros = jnp.zeros((_L,), jnp.float32)
        for k in range(_HREP):
            for r in range(_BINS_PAD):
                hcnts[k][pl.ds(r * _L, _L)] = zeros
                hsums[k][pl.ds(r * _L, _L)] = zeros

        def copies(j, slot):
            row0 = base + j * cr
            return pltpu.make_async_copy(
                lv_hbm.at[pl.ds(row0, cr), :], bufs[slot], sems[slot])

        copies(0, 0).start()

        for j in range(nchunks):
            slot = j % 2
            if j + 1 < nchunks:
                copies(j + 1, 1 - slot).start()
            copies(j, slot).wait()
            blv = bufs[slot]

            def step(r, carry):
                vals = []
                for c in range(128 // _L):
                    vb = blv[r, pl.ds(c * _L, _L)]
                    slot_i = ((vb & 15) << 4) + lane
                    lossv = lax.bitcast_convert_type(vb & ~15, jnp.float32)
                    vals.append((c % _HREP, slot_i, lossv))
                for k, slot_i, lossv in vals:
                    plsc.addupdate_scatter(hsums[k], [slot_i], lossv)
                for k, slot_i, _ in vals:
                    plsc.addupdate_scatter(hcnts[k], [slot_i], ones)
                return carry

            lax.fori_loop(0, cr, step, 0, unroll=2)

        for k in range(_HREP):
            pltpu.sync_copy(hcnts[k], cnt_out.at[wid, pl.ds(k * hwords, hwords)])
            pltpu.sync_copy(hsums[k], sum_out.at[wid, pl.ds(k * hwords, hwords)])

    return body


def _epilogue_body(cnt_ref, sum_ref, out_ref):
    c = jnp.sum(cnt_ref[...], axis=0)
    s = jnp.sum(sum_ref[...], axis=0)
    cb = jnp.sum(c, axis=1, keepdims=True)
    sb = jnp.sum(s, axis=1, keepdims=True)
    rowid = lax.broadcasted_iota(jnp.int32, cb.shape, 0)
    nz = (cb > 0.0) & (rowid < _BINS)
    n = jnp.sum(nz.astype(jnp.float32))
    denom = jnp.where(nz, cb * n, 1.0)
    contrib = jnp.where(nz, sb / denom, 0.0)
    total = jnp.sum(contrib, keepdims=True) * jnp.float32(_LOSS_WEIGHT)
    out_ref[...] = total.reshape(1, 1)


def kernel(pred, target, label_weight):
    n_rows, ncol = pred.shape
    nb = -(-n_rows // _BW)
    rb = ncol * _BW // 128
    while (nb * rb) % (2 * _NW * 16):
        nb += 1
    out_r = nb * rb

    xt = pred.T
    tt = target.T
    wt = label_weight.T

    packed = pl.pallas_call(
        _make_elem_body(n_rows, _BW, nb),
        grid=(nb,),
        in_specs=[pl.BlockSpec((ncol, _BW), lambda b: (0, b))] * 3,
        out_specs=[pl.BlockSpec((rb, 128), lambda b: (b, 0))],
        out_shape=[jax.ShapeDtypeStruct((out_r, 128), jnp.int32)],
    )(xt, tt, wt)[0]

    cr = out_r // _NW // 16
    mesh = plsc.VectorSubcoreMesh(core_axis_name="c", subcore_axis_name="s")
    hist = pl.kernel(
        _make_hist_body(out_r),
        out_type=(
            jax.ShapeDtypeStruct((_NW, _HREP * _BINS_PAD * _L), jnp.float32),
            jax.ShapeDtypeStruct((_NW, _HREP * _BINS_PAD * _L), jnp.float32),
        ),
        mesh=mesh,
        scratch_types=(
            tuple(pltpu.VMEM((cr, 128), jnp.int32) for _ in range(2)),
            tuple(pltpu.VMEM((_BINS_PAD * _L,), jnp.float32)
                  for _ in range(_HREP)),
            tuple(pltpu.VMEM((_BINS_PAD * _L,), jnp.float32)
                  for _ in range(_HREP)),
            tuple(pltpu.SemaphoreType.DMA for _ in range(2)),
        ),
        compiler_params=pltpu.CompilerParams(
            needs_layout_passes=False, use_tc_tiling_on_sc=False),
    )
    cnt, sums = hist(packed)
    cnt = cnt.reshape(_NW * _HREP, _BINS_PAD, _L)
    sums = sums.reshape(_NW * _HREP, _BINS_PAD, _L)

    out = pl.pallas_call(
        _epilogue_body,
        out_shape=jax.ShapeDtypeStruct((1, 1), jnp.float32),
    )(cnt, sums)
    return out[0, 0]
